# Initial kernel scaffold; baseline (speedup 1.0000x reference)
import jax
import jax.numpy as jnp
from jax.experimental import pallas as pl


def kernel(x):
    raise NotImplementedError("write your pallas kernel here")


# speedup vs baseline: 485.7115x; 485.7115x over previous
import functools

import jax
import jax.numpy as jnp
from jax import lax
from jax.experimental import pallas as pl
from jax.experimental.pallas import tpu as pltpu
from jax.experimental.pallas import tpu_sc as plsc

BATCH = 16384
N_IN = 16
N_RULES = 64
NUM_CORES = 2
NUM_SUBCORES = 16
NUM_WORKERS = NUM_CORES * NUM_SUBCORES
ROWS_PER_WORKER = BATCH // NUM_WORKERS
LANES = 16
BLOCKS = ROWS_PER_WORKER // LANES


def _body(x_hbm, out_hbm, in_v, out_v):
    wid = lax.axis_index("s") * NUM_CORES + lax.axis_index("c")
    base = wid * ROWS_PER_WORKER

    pltpu.sync_copy(x_hbm.at[pl.ds(base, ROWS_PER_WORKER)], in_v)

    iota = lax.iota(jnp.int32, LANES)

    def block(t, carry):
        row_idx = t * LANES + iota
        cols = [
            plsc.load_gather(in_v, [row_idx, jnp.full((LANES,), i, jnp.int32)])
            for i in range(N_IN)
        ]
        a = cols[:8]
        c = cols[8:]
        for i in range(8):
            for j in range(8):
                plsc.store_scatter(
                    out_v,
                    [row_idx, jnp.full((LANES,), i * 8 + j, jnp.int32)],
                    jnp.minimum(a[i], c[j]),
                )
        return carry

    lax.fori_loop(0, BLOCKS, block, 0)

    pltpu.sync_copy(out_v, out_hbm.at[pl.ds(base, ROWS_PER_WORKER)])


@functools.partial(jax.jit, static_argnames=())
def _run(x2):
    mesh = plsc.VectorSubcoreMesh(
        core_axis_name="c", subcore_axis_name="s",
        num_cores=NUM_CORES, num_subcores=NUM_SUBCORES,
    )
    k = pl.kernel(
        _body,
        out_type=jax.ShapeDtypeStruct((BATCH, N_RULES), jnp.float32),
        mesh=mesh,
        scratch_types=[
            pltpu.VMEM((ROWS_PER_WORKER, N_IN), jnp.float32),
            pltpu.VMEM((ROWS_PER_WORKER, N_RULES), jnp.float32),
        ],
        compiler_params=pltpu.CompilerParams(needs_layout_passes=False),
    )
    return k(x2)


def kernel(x):
    x2 = x.reshape(BATCH, N_IN)
    return _run(x2)


# speedup vs baseline: 699.5925x; 1.4403x over previous
import functools

import jax
import jax.numpy as jnp
from jax import lax
from jax.experimental import pallas as pl
from jax.experimental.pallas import tpu as pltpu
from jax.experimental.pallas import tpu_sc as plsc

BATCH = 16384
N_IN = 16
N_RULES = 64
NUM_CORES = 2
NUM_SUBCORES = 16
NUM_WORKERS = NUM_CORES * NUM_SUBCORES
ROWS_PER_WORKER = BATCH // NUM_WORKERS
LANES = 16
ROWS_PER_BLOCK = 16
BLOCKS = ROWS_PER_WORKER // ROWS_PER_BLOCK

_GATHER_DNUMS = lax.GatherDimensionNumbers(
    offset_dims=(), collapsed_slice_dims=(0,), start_index_map=(0,))


def _perm(v, idx):
    return lax.gather(v, idx[:, None], _GATHER_DNUMS, slice_sizes=(1,),
                      mode=lax.GatherScatterMode.PROMISE_IN_BOUNDS)


def _body(x_hbm, out_hbm, in_v, out_v):
    wid = lax.axis_index("s") * NUM_CORES + lax.axis_index("c")
    base = wid * ROWS_PER_WORKER

    pltpu.sync_copy(x_hbm.at[pl.ds(base, ROWS_PER_WORKER)], in_v)

    iota = lax.iota(jnp.int32, LANES)
    idx_c = 8 + jnp.bitwise_and(iota, 7)
    idx_a = [2 * k + jnp.right_shift(iota, 3) for k in range(4)]

    def block(t, carry):
        row0 = t * ROWS_PER_BLOCK
        for r in range(ROWS_PER_BLOCK):
            row = row0 + r
            v = in_v[row, :]
            c = _perm(v, idx_c)
            for k in range(4):
                a = _perm(v, idx_a[k])
                out_v[row, pl.ds(16 * k, 16)] = jnp.minimum(a, c)
        return carry

    lax.fori_loop(0, BLOCKS, block, 0)

    pltpu.sync_copy(out_v, out_hbm.at[pl.ds(base, ROWS_PER_WORKER)])


@functools.partial(jax.jit, static_argnames=())
def _run(x2):
    mesh = plsc.VectorSubcoreMesh(
        core_axis_name="c", subcore_axis_name="s",
        num_cores=NUM_CORES, num_subcores=NUM_SUBCORES,
    )
    k = pl.kernel(
        _body,
        out_type=jax.ShapeDtypeStruct((BATCH, N_RULES), jnp.float32),
        mesh=mesh,
        scratch_types=[
            pltpu.VMEM((ROWS_PER_WORKER, N_IN), jnp.float32),
            pltpu.VMEM((ROWS_PER_WORKER, N_RULES), jnp.float32),
        ],
        compiler_params=pltpu.CompilerParams(needs_layout_passes=False),
    )
    return k(x2)


def kernel(x):
    x2 = x.reshape(BATCH, N_IN)
    return _run(x2)
